# baseline (speedup 1.0000x reference)
import functools

import jax
import jax.numpy as jnp
from jax import lax
from jax.experimental import pallas as pl
from jax.experimental.pallas import tpu as pltpu
from jax.experimental.pallas import tpu_sc as plsc

M = 100000
F = 128
B = 4096
C = 1000
TEMP = 0.05

NC = 2
NS = 16
NW = NC * NS

CHUNK = 128
NFULL = M // CHUNK
TAIL = M - NFULL * CHUNK
MAXQ = -(-NFULL // NW)
C_PAD = 1024
B_PER_W = B // NW


@functools.cache
def _build_sc_segsum():
    mesh = plsc.VectorSubcoreMesh(core_axis_name="c", subcore_axis_name="s")

    @functools.partial(
        pl.kernel,
        mesh=mesh,
        out_type=(
            jax.ShapeDtypeStruct((NC, C_PAD, F), jnp.float32),
            jax.ShapeDtypeStruct((NC, C_PAD, 16), jnp.float32),
            jax.ShapeDtypeStruct((B,), jnp.int32),
        ),
        scratch_types=[
            pltpu.VMEM((2, CHUNK, F), jnp.float32),
            pltpu.VMEM((MAXQ, CHUNK), jnp.int32),
            pltpu.VMEM((TAIL,), jnp.int32),
            pltpu.VMEM((CHUNK, 16), jnp.float32),
            pltpu.VMEM((B_PER_W,), jnp.int32),
            pltpu.VMEM((B_PER_W,), jnp.int32),
            pltpu.VMEM_SHARED((C_PAD, F), jnp.float32),
            pltpu.VMEM_SHARED((C_PAD, 16), jnp.float32),
            pltpu.SemaphoreType.DMA((2,)),
            pltpu.SemaphoreType.DMA((2,)),
            pltpu.SemaphoreType.DMA((2,)),
        ],
    )
    def sc_segsum(feat_hbm, lblq_hbm, lbl_hbm, idx_hbm, ones_hbm, zg_hbm,
                  zn_hbm, partials_hbm, counts_hbm, targets_hbm,
                  feat_v, lbl_v, ltail_v, ones_v, idx_v, tgt_v,
                  acc_g, acc_n, sem_f, sem_s, sem_o):
        c = lax.axis_index("c")
        s = lax.axis_index("s")
        w = s * NC + c

        @pl.when(s == 0)
        def _():
            pltpu.sync_copy(zg_hbm, acc_g)
            pltpu.sync_copy(zn_hbm, acc_n)

        pltpu.sync_copy(ones_hbm, ones_v)
        pltpu.sync_copy(lblq_hbm.at[w], lbl_v)
        plsc.subcore_barrier()

        def feat_off(i):
            return jnp.minimum(i * NW + w, NFULL - 1) * CHUNK

        def load(i, slot):
            pltpu.async_copy(
                feat_hbm.at[pl.ds(feat_off(i), CHUNK)], feat_v.at[slot],
                sem_f.at[slot])

        def wait_load(i, slot):
            pltpu.make_async_copy(
                feat_hbm.at[pl.ds(feat_off(i), CHUNK)], feat_v.at[slot],
                sem_f.at[slot]).wait()

        def start_scatter(i, slot):
            pltpu.async_copy(
                feat_v.at[slot], acc_g.at[lbl_v.at[i]], sem_s.at[slot],
                add=True)
            pltpu.async_copy(
                ones_v, acc_n.at[lbl_v.at[i]], sem_o.at[slot], add=True)

        def wait_scatter(i, slot):
            pltpu.make_async_copy(
                feat_v.at[slot], acc_g.at[lbl_v.at[i]], sem_s.at[slot]).wait()
            pltpu.make_async_copy(
                ones_v, acc_n.at[lbl_v.at[i]], sem_o.at[slot]).wait()

        load(0, 0)
        for i in range(MAXQ):
            slot = i & 1
            wait_load(i, slot)
            if i >= 1:
                wait_scatter(i - 1, 1 - slot)
            if i + 1 < MAXQ:
                load(i + 1, 1 - slot)
            start_scatter(i, slot)
        wait_scatter(MAXQ - 1, (MAXQ - 1) & 1)

        @pl.when(w == NW - 1)
        def _():
            off = NFULL * CHUNK
            pltpu.sync_copy(feat_hbm.at[pl.ds(off, TAIL)],
                            feat_v.at[0].at[pl.ds(0, TAIL)])
            pltpu.sync_copy(lbl_hbm.at[pl.ds(off, TAIL)], ltail_v)
            pltpu.sync_copy(feat_v.at[0].at[pl.ds(0, TAIL)],
                            acc_g.at[ltail_v], add=True)
            pltpu.sync_copy(ones_v.at[pl.ds(0, TAIL)],
                            acc_n.at[ltail_v], add=True)

        pltpu.sync_copy(idx_hbm.at[pl.ds(w * B_PER_W, B_PER_W)], idx_v)
        pltpu.sync_copy(lbl_hbm.at[idx_v], tgt_v)
        pltpu.sync_copy(tgt_v, targets_hbm.at[pl.ds(w * B_PER_W, B_PER_W)])

        plsc.subcore_barrier()

        @pl.when(s == 0)
        def _():
            pltpu.sync_copy(acc_g, partials_hbm.at[c])
            pltpu.sync_copy(acc_n, counts_hbm.at[c])

    return sc_segsum


_BLK = 512
_GRID = B // _BLK


def _tc_body(res_ref, part_ref, nums_ref, tgt_ref, out_ref, g_ref, acc):
    i = pl.program_id(0)

    @pl.when(i == 0)
    def _():
        g_ref[...] = part_ref[0] + part_ref[1]
        acc[0] = 0.0

    x = res_ref[...]
    norm = jnp.sqrt(jnp.sum(x * x, axis=1, keepdims=True))
    x = x / jnp.maximum(norm, 1e-12)

    vec = lax.dot_general(x, g_ref[...], (((1,), (1,)), ((), ())),
                          preferred_element_type=jnp.float32,
                          precision=lax.Precision.HIGHEST)

    nums = nums_ref[...]
    col = lax.broadcasted_iota(jnp.int32, (1, C_PAD), 1)
    mask = jnp.where((nums > 0.0) & (col < C), 1.0, 0.0)
    scale = 1.0 / (TEMP * jnp.maximum(nums, 1.0))

    mexp = jnp.exp(vec * scale) * mask
    sums = jnp.sum(mexp, axis=1, keepdims=True) + 1e-6

    t = tgt_ref[...]
    hit = lax.broadcasted_iota(jnp.int32, (_BLK, C_PAD), 1) == t
    ex_t = jnp.sum(jnp.where(hit, mexp, 0.0), axis=1, keepdims=True)
    picked = jnp.log(ex_t + 1e-6 * sums) - jnp.log(sums)

    acc[0] += jnp.sum(picked)

    @pl.when(i == _GRID - 1)
    def _():
        out_ref[...] = jnp.full((1, 1), -acc[0] / float(B), jnp.float32)


def _tc_loss(results, partials, nums_row, targets_col):
    return pl.pallas_call(
        _tc_body,
        grid=(_GRID,),
        in_specs=[
            pl.BlockSpec((_BLK, F), lambda i: (i, 0)),
            pl.BlockSpec((NC, C_PAD, F), lambda i: (0, 0, 0)),
            pl.BlockSpec((1, C_PAD), lambda i: (0, 0)),
            pl.BlockSpec((_BLK, 1), lambda i: (i, 0)),
        ],
        out_specs=pl.BlockSpec((1, 1), lambda i: (0, 0)),
        out_shape=jax.ShapeDtypeStruct((1, 1), jnp.float32),
        scratch_shapes=[
            pltpu.VMEM((C_PAD, F), jnp.float32),
            pltpu.SMEM((1,), jnp.float32),
        ],
    )(results, partials, nums_row, targets_col)


def kernel(results, indexes, features, labels):
    ones_rows = jnp.ones((CHUNK, 16), jnp.float32)
    zg = jnp.zeros((C_PAD, F), jnp.float32)
    zn = jnp.zeros((C_PAD, 16), jnp.float32)
    lbl_pad = jnp.concatenate(
        [labels[:NFULL * CHUNK],
         jnp.full((MAXQ * NW * CHUNK - NFULL * CHUNK,), C, jnp.int32)])
    lblq = lbl_pad.reshape(MAXQ, NW, CHUNK).transpose(1, 0, 2)

    partials, counts, targets = _build_sc_segsum()(
        features, lblq, labels, indexes, ones_rows, zg, zn)

    nums_row = (counts[0, :, 0] + counts[1, :, 0]).reshape(1, C_PAD)
    targets_col = targets.reshape(B, 1)

    loss = _tc_loss(results, partials, nums_row, targets_col)
    return loss.reshape(())


# speedup vs baseline: 1.0207x; 1.0207x over previous
import functools

import jax
import jax.numpy as jnp
from jax import lax
from jax.experimental import pallas as pl
from jax.experimental.pallas import tpu as pltpu
from jax.experimental.pallas import tpu_sc as plsc

M = 100000
F = 128
B = 4096
C = 1000
TEMP = 0.05

NC = 2
NS = 16
NW = NC * NS

CHUNK = 128
NFULL = M // CHUNK
TAIL = M - NFULL * CHUNK
MAXQ = -(-NFULL // NW)
C_PAD = 1024
B_PER_W = B // NW


@functools.cache
def _build_sc_segsum():
    mesh = plsc.VectorSubcoreMesh(core_axis_name="c", subcore_axis_name="s")

    @functools.partial(
        pl.kernel,
        mesh=mesh,
        out_type=(
            jax.ShapeDtypeStruct((NC, C_PAD, F), jnp.float32),
            jax.ShapeDtypeStruct((NC, C_PAD, 16), jnp.float32),
            jax.ShapeDtypeStruct((B,), jnp.int32),
        ),
        scratch_types=[
            pltpu.VMEM((2, CHUNK, F), jnp.float32),
            pltpu.VMEM((MAXQ, CHUNK), jnp.int32),
            pltpu.VMEM((TAIL,), jnp.int32),
            pltpu.VMEM((CHUNK, 16), jnp.float32),
            pltpu.VMEM((B_PER_W,), jnp.int32),
            pltpu.VMEM((B_PER_W,), jnp.int32),
            pltpu.VMEM_SHARED((C_PAD, F), jnp.float32),
            pltpu.VMEM_SHARED((C_PAD, 16), jnp.float32),
            pltpu.SemaphoreType.DMA((2,)),
            pltpu.SemaphoreType.DMA((2,)),
            pltpu.SemaphoreType.DMA((2,)),
        ],
    )
    def sc_segsum(feat_hbm, lblq_hbm, lbl_hbm, idx_hbm, ones_hbm, zg_hbm,
                  zn_hbm, partials_hbm, counts_hbm, targets_hbm,
                  feat_v, lbl_v, ltail_v, ones_v, idx_v, tgt_v,
                  acc_g, acc_n, sem_f, sem_s, sem_o):
        c = lax.axis_index("c")
        s = lax.axis_index("s")
        w = s * NC + c

        @pl.when(s == 0)
        def _():
            pltpu.sync_copy(zg_hbm, acc_g)
            pltpu.sync_copy(zn_hbm, acc_n)

        pltpu.sync_copy(ones_hbm, ones_v)
        pltpu.sync_copy(lblq_hbm.at[w], lbl_v)
        plsc.subcore_barrier()

        def feat_off(i):
            return jnp.minimum(i * NW + w, NFULL - 1) * CHUNK

        def load(i, slot):
            pltpu.async_copy(
                feat_hbm.at[pl.ds(feat_off(i), CHUNK)], feat_v.at[slot],
                sem_f.at[slot])

        def wait_load(i, slot):
            pltpu.make_async_copy(
                feat_hbm.at[pl.ds(feat_off(i), CHUNK)], feat_v.at[slot],
                sem_f.at[slot]).wait()

        def start_scatter(i, slot):
            pltpu.async_copy(
                feat_v.at[slot], acc_g.at[lbl_v.at[i]], sem_s.at[slot],
                add=True)
            pltpu.async_copy(
                ones_v, acc_n.at[lbl_v.at[i]], sem_o.at[slot], add=True)

        def wait_scatter(i, slot):
            pltpu.make_async_copy(
                feat_v.at[slot], acc_g.at[lbl_v.at[i]], sem_s.at[slot]).wait()
            pltpu.make_async_copy(
                ones_v, acc_n.at[lbl_v.at[i]], sem_o.at[slot]).wait()

        load(0, 0)
        for i in range(MAXQ):
            slot = i & 1
            wait_load(i, slot)
            if i + 1 < MAXQ:
                load(i + 1, 1 - slot)
            start_scatter(i, slot)
            wait_scatter(i, slot)

        @pl.when(w == NW - 1)
        def _():
            off = NFULL * CHUNK
            pltpu.sync_copy(feat_hbm.at[pl.ds(off, TAIL)],
                            feat_v.at[0].at[pl.ds(0, TAIL)])
            pltpu.sync_copy(lbl_hbm.at[pl.ds(off, TAIL)], ltail_v)
            pltpu.sync_copy(feat_v.at[0].at[pl.ds(0, TAIL)],
                            acc_g.at[ltail_v], add=True)
            pltpu.sync_copy(ones_v.at[pl.ds(0, TAIL)],
                            acc_n.at[ltail_v], add=True)

        pltpu.sync_copy(idx_hbm.at[pl.ds(w * B_PER_W, B_PER_W)], idx_v)
        pltpu.sync_copy(lbl_hbm.at[idx_v], tgt_v)
        pltpu.sync_copy(tgt_v, targets_hbm.at[pl.ds(w * B_PER_W, B_PER_W)])

        plsc.subcore_barrier()

        @pl.when(s == 0)
        def _():
            pltpu.sync_copy(acc_g, partials_hbm.at[c])
            pltpu.sync_copy(acc_n, counts_hbm.at[c])

    return sc_segsum


_BLK = 512
_GRID = B // _BLK


def _tc_body(res_ref, part_ref, cnt_ref, tgt_ref, out_ref, g_ref, nums_ref,
             acc):
    i = pl.program_id(0)

    @pl.when(i == 0)
    def _():
        g_ref[...] = part_ref[0] + part_ref[1]
        cnt = cnt_ref[0] + cnt_ref[1]
        sel = jnp.full((1, 16), 1.0 / 16.0, jnp.float32)
        nums_ref[...] = lax.dot_general(
            sel, cnt, (((1,), (1,)), ((), ())),
            preferred_element_type=jnp.float32,
            precision=lax.Precision.HIGHEST)
        acc[0] = 0.0

    x = res_ref[...]
    norm = jnp.sqrt(jnp.sum(x * x, axis=1, keepdims=True))
    x = x / jnp.maximum(norm, 1e-12)

    vec = lax.dot_general(x, g_ref[...], (((1,), (1,)), ((), ())),
                          preferred_element_type=jnp.float32,
                          precision=lax.Precision.HIGHEST)

    nums = nums_ref[...]
    col = lax.broadcasted_iota(jnp.int32, (1, C_PAD), 1)
    mask = jnp.where((nums > 0.0) & (col < C), 1.0, 0.0)
    scale = 1.0 / (TEMP * jnp.maximum(nums, 1.0))

    mexp = jnp.exp(vec * scale) * mask
    sums = jnp.sum(mexp, axis=1, keepdims=True) + 1e-6

    t = tgt_ref[...]
    hit = lax.broadcasted_iota(jnp.int32, (_BLK, C_PAD), 1) == t
    ex_t = jnp.sum(jnp.where(hit, mexp, 0.0), axis=1, keepdims=True)
    picked = jnp.log(ex_t + 1e-6 * sums) - jnp.log(sums)

    acc[0] += jnp.sum(picked)

    @pl.when(i == _GRID - 1)
    def _():
        out_ref[...] = jnp.full((1, 1), -acc[0] / float(B), jnp.float32)


def _tc_loss(results, partials, counts, targets_col):
    return pl.pallas_call(
        _tc_body,
        grid=(_GRID,),
        in_specs=[
            pl.BlockSpec((_BLK, F), lambda i: (i, 0)),
            pl.BlockSpec((NC, C_PAD, F), lambda i: (0, 0, 0)),
            pl.BlockSpec((NC, C_PAD, 16), lambda i: (0, 0, 0)),
            pl.BlockSpec((_BLK, 1), lambda i: (i, 0)),
        ],
        out_specs=pl.BlockSpec((1, 1), lambda i: (0, 0)),
        out_shape=jax.ShapeDtypeStruct((1, 1), jnp.float32),
        scratch_shapes=[
            pltpu.VMEM((C_PAD, F), jnp.float32),
            pltpu.VMEM((1, C_PAD), jnp.float32),
            pltpu.SMEM((1,), jnp.float32),
        ],
    )(results, partials, counts, targets_col)


def kernel(results, indexes, features, labels):
    ones_rows = jnp.ones((CHUNK, 16), jnp.float32)
    zg = jnp.zeros((C_PAD, F), jnp.float32)
    zn = jnp.zeros((C_PAD, 16), jnp.float32)
    npad = MAXQ * NW * CHUNK - NFULL * CHUNK
    dummy = C + (jnp.arange(npad, dtype=jnp.int32) % (C_PAD - C))
    lbl_pad = jnp.concatenate([labels[:NFULL * CHUNK], dummy])
    lblq = lbl_pad.reshape(MAXQ, NW, CHUNK).transpose(1, 0, 2)

    partials, counts, targets = _build_sc_segsum()(
        features, lblq, labels, indexes, ones_rows, zg, zn)

    targets_col = targets.reshape(B, 1)

    loss = _tc_loss(results, partials, counts, targets_col)
    return loss.reshape(())


# speedup vs baseline: 1.1495x; 1.1262x over previous
import functools

import jax
import jax.numpy as jnp
from jax import lax
from jax.experimental import pallas as pl
from jax.experimental.pallas import tpu as pltpu
from jax.experimental.pallas import tpu_sc as plsc

M = 100000
F = 128
B = 4096
C = 1000
TEMP = 0.05

NC = 2
NS = 16
NW = NC * NS

CHUNK = 128
NFULL = M // CHUNK
TAIL = M - NFULL * CHUNK
MAXQ = -(-NFULL // NW)
C_PAD = 1024
B_PER_W = B // NW


@functools.cache
def _build_sc_segsum():
    mesh = plsc.VectorSubcoreMesh(core_axis_name="c", subcore_axis_name="s")

    @functools.partial(
        pl.kernel,
        mesh=mesh,
        out_type=(
            jax.ShapeDtypeStruct((NC, C_PAD, F), jnp.float32),
            jax.ShapeDtypeStruct((NC, C_PAD, 16), jnp.float32),
            jax.ShapeDtypeStruct((B,), jnp.int32),
        ),
        scratch_types=[
            pltpu.VMEM((2, CHUNK, F), jnp.float32),
            pltpu.VMEM((MAXQ, CHUNK), jnp.int32),
            pltpu.VMEM((TAIL,), jnp.int32),
            pltpu.VMEM((CHUNK, 16), jnp.float32),
            pltpu.VMEM((B_PER_W,), jnp.int32),
            pltpu.VMEM((B_PER_W,), jnp.int32),
            pltpu.VMEM_SHARED((C_PAD, F), jnp.float32),
            pltpu.VMEM_SHARED((C_PAD, 16), jnp.float32),
            pltpu.SemaphoreType.DMA((2,)),
            pltpu.SemaphoreType.DMA((2,)),
            pltpu.SemaphoreType.DMA((2,)),
        ],
    )
    def sc_segsum(feat_hbm, lblq_hbm, lbl_hbm, idx_hbm, ones_hbm, zg_hbm,
                  zn_hbm, partials_hbm, counts_hbm, targets_hbm,
                  feat_v, lbl_v, ltail_v, ones_v, idx_v, tgt_v,
                  acc_g, acc_n, sem_f, sem_s, sem_o):
        c = lax.axis_index("c")
        s = lax.axis_index("s")
        w = s * NC + c

        @pl.when(s == 0)
        def _():
            pltpu.sync_copy(zg_hbm, acc_g)
            pltpu.sync_copy(zn_hbm, acc_n)

        pltpu.sync_copy(ones_hbm, ones_v)
        pltpu.sync_copy(lblq_hbm.at[w], lbl_v)
        plsc.subcore_barrier()

        def feat_off(i):
            return jnp.minimum(i * NW + w, NFULL - 1) * CHUNK

        def load(i, slot):
            pltpu.async_copy(
                feat_hbm.at[pl.ds(feat_off(i), CHUNK)], feat_v.at[slot],
                sem_f.at[slot])

        def wait_load(i, slot):
            pltpu.make_async_copy(
                feat_hbm.at[pl.ds(feat_off(i), CHUNK)], feat_v.at[slot],
                sem_f.at[slot]).wait()

        def start_scatter(i, slot):
            pltpu.async_copy(
                feat_v.at[slot], acc_g.at[lbl_v.at[i]], sem_s.at[slot],
                add=True)
            pltpu.async_copy(
                ones_v, acc_n.at[lbl_v.at[i]], sem_o.at[slot], add=True)

        def wait_scatter(i, slot):
            pltpu.make_async_copy(
                feat_v.at[slot], acc_g.at[lbl_v.at[i]], sem_s.at[slot]).wait()
            pltpu.make_async_copy(
                ones_v, acc_n.at[lbl_v.at[i]], sem_o.at[slot]).wait()

        load(0, 0)
        for i in range(MAXQ):
            slot = i & 1
            wait_load(i, slot)
            if i + 1 < MAXQ:
                load(i + 1, 1 - slot)
            start_scatter(i, slot)
            wait_scatter(i, slot)

        @pl.when(w == NW - 1)
        def _():
            off = NFULL * CHUNK
            pltpu.sync_copy(feat_hbm.at[pl.ds(off, TAIL)],
                            feat_v.at[0].at[pl.ds(0, TAIL)])
            pltpu.sync_copy(lbl_hbm.at[pl.ds(off, TAIL)], ltail_v)
            pltpu.sync_copy(feat_v.at[0].at[pl.ds(0, TAIL)],
                            acc_g.at[ltail_v], add=True)
            pltpu.sync_copy(ones_v.at[pl.ds(0, TAIL)],
                            acc_n.at[ltail_v], add=True)

        pltpu.sync_copy(idx_hbm.at[pl.ds(w * B_PER_W, B_PER_W)], idx_v)
        pltpu.sync_copy(lbl_hbm.at[idx_v], tgt_v)
        pltpu.sync_copy(tgt_v, targets_hbm.at[pl.ds(w * B_PER_W, B_PER_W)])

        plsc.subcore_barrier()

        @pl.when(s == 0)
        def _():
            pltpu.sync_copy(acc_g, partials_hbm.at[c])
            pltpu.sync_copy(acc_n, counts_hbm.at[c])

    return sc_segsum


_BLK = 512
_GRID = B // _BLK


def _tc_body(res_ref, part_ref, cnt_ref, tgt_ref, out_ref, g_ref, nums_ref,
             acc):
    i = pl.program_id(0)

    @pl.when(i == 0)
    def _():
        g_ref[...] = part_ref[0] + part_ref[1]
        cnt = cnt_ref[0] + cnt_ref[1]
        sel = jnp.full((1, 16), 1.0 / 16.0, jnp.float32)
        nums_ref[...] = lax.dot_general(
            sel, cnt, (((1,), (1,)), ((), ())),
            preferred_element_type=jnp.float32,
            precision=lax.Precision.HIGHEST)
        acc[0] = 0.0

    x = res_ref[...]
    norm = jnp.sqrt(jnp.sum(x * x, axis=1, keepdims=True))
    x = x / jnp.maximum(norm, 1e-12)

    vec = lax.dot_general(x, g_ref[...], (((1,), (1,)), ((), ())),
                          preferred_element_type=jnp.float32)

    nums = nums_ref[...]
    col = lax.broadcasted_iota(jnp.int32, (1, C_PAD), 1)
    mask = jnp.where((nums > 0.0) & (col < C), 1.0, 0.0)
    scale = 1.0 / (TEMP * jnp.maximum(nums, 1.0))

    mexp = jnp.exp(vec * scale) * mask
    sums = jnp.sum(mexp, axis=1, keepdims=True) + 1e-6

    t = tgt_ref[...]
    hit = lax.broadcasted_iota(jnp.int32, (_BLK, C_PAD), 1) == t
    ex_t = jnp.sum(jnp.where(hit, mexp, 0.0), axis=1, keepdims=True)
    picked = jnp.log(ex_t + 1e-6 * sums) - jnp.log(sums)

    acc[0] += jnp.sum(picked)

    @pl.when(i == _GRID - 1)
    def _():
        out_ref[...] = jnp.full((1, 1), -acc[0] / float(B), jnp.float32)


def _tc_loss(results, partials, counts, targets_col):
    return pl.pallas_call(
        _tc_body,
        grid=(_GRID,),
        in_specs=[
            pl.BlockSpec((_BLK, F), lambda i: (i, 0)),
            pl.BlockSpec((NC, C_PAD, F), lambda i: (0, 0, 0)),
            pl.BlockSpec((NC, C_PAD, 16), lambda i: (0, 0, 0)),
            pl.BlockSpec((_BLK, 1), lambda i: (i, 0)),
        ],
        out_specs=pl.BlockSpec((1, 1), lambda i: (0, 0)),
        out_shape=jax.ShapeDtypeStruct((1, 1), jnp.float32),
        scratch_shapes=[
            pltpu.VMEM((C_PAD, F), jnp.float32),
            pltpu.VMEM((1, C_PAD), jnp.float32),
            pltpu.SMEM((1,), jnp.float32),
        ],
    )(results, partials, counts, targets_col)


def kernel(results, indexes, features, labels):
    ones_rows = jnp.ones((CHUNK, 16), jnp.float32)
    zg = jnp.zeros((C_PAD, F), jnp.float32)
    zn = jnp.zeros((C_PAD, 16), jnp.float32)
    npad = MAXQ * NW * CHUNK - NFULL * CHUNK
    dummy = C + (jnp.arange(npad, dtype=jnp.int32) % (C_PAD - C))
    lbl_pad = jnp.concatenate([labels[:NFULL * CHUNK], dummy])
    lblq = lbl_pad.reshape(MAXQ, NW, CHUNK).transpose(1, 0, 2)

    partials, counts, targets = _build_sc_segsum()(
        features, lblq, labels, indexes, ones_rows, zg, zn)

    targets_col = targets.reshape(B, 1)

    loss = _tc_loss(results, partials, counts, targets_col)
    return loss.reshape(())


# speedup vs baseline: 1.2455x; 1.0835x over previous
import functools

import jax
import jax.numpy as jnp
from jax import lax
from jax.experimental import pallas as pl
from jax.experimental.pallas import tpu as pltpu
from jax.experimental.pallas import tpu_sc as plsc

M = 100000
F = 128
B = 4096
C = 1000
TEMP = 0.05

NC = 2
NS = 16
NW = NC * NS

CHUNK = 128
NFULL = M // CHUNK
TAIL = M - NFULL * CHUNK
MAXQ = -(-NFULL // NW)
C_PAD = 1024
B_PER_W = B // NW


@functools.cache
def _build_sc_segsum():
    mesh = plsc.VectorSubcoreMesh(core_axis_name="c", subcore_axis_name="s")

    @functools.partial(
        pl.kernel,
        mesh=mesh,
        out_type=(
            jax.ShapeDtypeStruct((NC, C_PAD, F), jnp.float32),
            jax.ShapeDtypeStruct((NC, C_PAD, 16), jnp.float32),
            jax.ShapeDtypeStruct((B,), jnp.int32),
        ),
        scratch_types=[
            pltpu.VMEM((2, CHUNK, F), jnp.float32),
            pltpu.VMEM((MAXQ, CHUNK), jnp.int32),
            pltpu.VMEM((TAIL,), jnp.int32),
            pltpu.VMEM((CHUNK, 16), jnp.float32),
            pltpu.VMEM((B_PER_W,), jnp.int32),
            pltpu.VMEM((B_PER_W,), jnp.int32),
            pltpu.VMEM_SHARED((C_PAD, F), jnp.float32),
            pltpu.VMEM_SHARED((C_PAD, 16), jnp.float32),
            pltpu.SemaphoreType.DMA((2,)),
            pltpu.SemaphoreType.DMA((2,)),
            pltpu.SemaphoreType.DMA((2,)),
        ],
    )
    def sc_segsum(feat_hbm, lblq_hbm, lbl_hbm, idx_hbm, ones_hbm, zg_hbm,
                  zn_hbm, partials_hbm, counts_hbm, targets_hbm,
                  feat_v, lbl_v, ltail_v, ones_v, idx_v, tgt_v,
                  acc_g, acc_n, sem_f, sem_s, sem_o):
        c = lax.axis_index("c")
        s = lax.axis_index("s")
        w = s * NC + c

        zrows = C_PAD // NS
        pltpu.sync_copy(zg_hbm.at[pl.ds(s * zrows, zrows)],
                        acc_g.at[pl.ds(s * zrows, zrows)])
        pltpu.sync_copy(zn_hbm.at[pl.ds(s * zrows, zrows)],
                        acc_n.at[pl.ds(s * zrows, zrows)])

        pltpu.sync_copy(ones_hbm, ones_v)
        pltpu.sync_copy(lblq_hbm.at[w], lbl_v)
        pltpu.sync_copy(idx_hbm.at[pl.ds(w * B_PER_W, B_PER_W)], idx_v)

        def feat_off(i):
            return jnp.minimum(i * NW + w, NFULL - 1) * CHUNK

        def load(i, slot):
            pltpu.async_copy(
                feat_hbm.at[pl.ds(feat_off(i), CHUNK)], feat_v.at[slot],
                sem_f.at[slot])

        def wait_load(i, slot):
            pltpu.make_async_copy(
                feat_hbm.at[pl.ds(feat_off(i), CHUNK)], feat_v.at[slot],
                sem_f.at[slot]).wait()

        def start_scatter(i, slot):
            pltpu.async_copy(
                feat_v.at[slot], acc_g.at[lbl_v.at[i]], sem_s.at[slot],
                add=True)
            pltpu.async_copy(
                ones_v, acc_n.at[lbl_v.at[i]], sem_o.at[slot], add=True)

        def wait_scatter(i, slot):
            pltpu.make_async_copy(
                feat_v.at[slot], acc_g.at[lbl_v.at[i]], sem_s.at[slot]).wait()
            pltpu.make_async_copy(
                ones_v, acc_n.at[lbl_v.at[i]], sem_o.at[slot]).wait()

        load(0, 0)
        load(1, 1)
        plsc.subcore_barrier()
        for i in range(MAXQ):
            slot = i & 1
            wait_load(i, slot)
            start_scatter(i, slot)
            wait_scatter(i, slot)
            if i + 2 < MAXQ:
                load(i + 2, slot)

        @pl.when(w == NW - 1)
        def _():
            off = NFULL * CHUNK
            pltpu.sync_copy(feat_hbm.at[pl.ds(off, TAIL)],
                            feat_v.at[0].at[pl.ds(0, TAIL)])
            pltpu.sync_copy(lbl_hbm.at[pl.ds(off, TAIL)], ltail_v)
            pltpu.sync_copy(feat_v.at[0].at[pl.ds(0, TAIL)],
                            acc_g.at[ltail_v], add=True)
            pltpu.sync_copy(ones_v.at[pl.ds(0, TAIL)],
                            acc_n.at[ltail_v], add=True)

        pltpu.sync_copy(lbl_hbm.at[idx_v], tgt_v)
        pltpu.sync_copy(tgt_v, targets_hbm.at[pl.ds(w * B_PER_W, B_PER_W)])

        plsc.subcore_barrier()

        @pl.when(s == 0)
        def _():
            pltpu.sync_copy(acc_g, partials_hbm.at[c])
            pltpu.sync_copy(acc_n, counts_hbm.at[c])

    return sc_segsum


_BLK = 512
_GRID = B // _BLK


def _tc_body(res_ref, part_ref, cnt_ref, tgt_ref, out_ref, g_ref, nums_ref,
             acc):
    i = pl.program_id(0)

    @pl.when(i == 0)
    def _():
        g_ref[...] = part_ref[0] + part_ref[1]
        cnt = cnt_ref[0] + cnt_ref[1]
        sel = jnp.full((1, 16), 1.0 / 16.0, jnp.float32)
        nums_ref[...] = lax.dot_general(
            sel, cnt, (((1,), (1,)), ((), ())),
            preferred_element_type=jnp.float32,
            precision=lax.Precision.HIGHEST)
        acc[0] = 0.0

    x = res_ref[...]
    norm = jnp.sqrt(jnp.sum(x * x, axis=1, keepdims=True))
    x = x / jnp.maximum(norm, 1e-12)

    vec = lax.dot_general(x, g_ref[...], (((1,), (1,)), ((), ())),
                          preferred_element_type=jnp.float32)

    nums = nums_ref[...]
    col = lax.broadcasted_iota(jnp.int32, (1, C_PAD), 1)
    mask = jnp.where((nums > 0.0) & (col < C), 1.0, 0.0)
    scale = 1.0 / (TEMP * jnp.maximum(nums, 1.0))

    mexp = jnp.exp(vec * scale) * mask
    sums = jnp.sum(mexp, axis=1, keepdims=True) + 1e-6

    t = tgt_ref[...]
    hit = lax.broadcasted_iota(jnp.int32, (_BLK, C_PAD), 1) == t
    ex_t = jnp.sum(jnp.where(hit, mexp, 0.0), axis=1, keepdims=True)
    picked = jnp.log(ex_t + 1e-6 * sums) - jnp.log(sums)

    acc[0] += jnp.sum(picked)

    @pl.when(i == _GRID - 1)
    def _():
        out_ref[...] = jnp.full((1, 1), -acc[0] / float(B), jnp.float32)


def _tc_loss(results, partials, counts, targets_col):
    return pl.pallas_call(
        _tc_body,
        grid=(_GRID,),
        in_specs=[
            pl.BlockSpec((_BLK, F), lambda i: (i, 0)),
            pl.BlockSpec((NC, C_PAD, F), lambda i: (0, 0, 0)),
            pl.BlockSpec((NC, C_PAD, 16), lambda i: (0, 0, 0)),
            pl.BlockSpec((_BLK, 1), lambda i: (i, 0)),
        ],
        out_specs=pl.BlockSpec((1, 1), lambda i: (0, 0)),
        out_shape=jax.ShapeDtypeStruct((1, 1), jnp.float32),
        scratch_shapes=[
            pltpu.VMEM((C_PAD, F), jnp.float32),
            pltpu.VMEM((1, C_PAD), jnp.float32),
            pltpu.SMEM((1,), jnp.float32),
        ],
    )(results, partials, counts, targets_col)


def kernel(results, indexes, features, labels):
    ones_rows = jnp.ones((CHUNK, 16), jnp.float32)
    zg = jnp.zeros((C_PAD, F), jnp.float32)
    zn = jnp.zeros((C_PAD, 16), jnp.float32)
    npad = MAXQ * NW * CHUNK - NFULL * CHUNK
    dummy = C + (jnp.arange(npad, dtype=jnp.int32) % (C_PAD - C))
    lbl_pad = jnp.concatenate([labels[:NFULL * CHUNK], dummy])
    lblq = lbl_pad.reshape(MAXQ, NW, CHUNK).transpose(1, 0, 2)

    partials, counts, targets = _build_sc_segsum()(
        features, lblq, labels, indexes, ones_rows, zg, zn)

    targets_col = targets.reshape(B, 1)

    loss = _tc_loss(results, partials, counts, targets_col)
    return loss.reshape(())


# speedup vs baseline: 1.2948x; 1.0396x over previous
import functools

import jax
import jax.numpy as jnp
from jax import lax
from jax.experimental import pallas as pl
from jax.experimental.pallas import tpu as pltpu
from jax.experimental.pallas import tpu_sc as plsc

M = 100000
F = 128
B = 4096
C = 1000
TEMP = 0.05

NC = 2
NS = 16
NW = NC * NS

CHUNK = 128
NFULL = M // CHUNK
TAIL = M - NFULL * CHUNK
MAXQ = -(-NFULL // NW)
C_PAD = 1024
B_PER_W = B // NW


@functools.cache
def _build_sc_segsum():
    mesh = plsc.VectorSubcoreMesh(core_axis_name="c", subcore_axis_name="s")

    @functools.partial(
        pl.kernel,
        mesh=mesh,
        out_type=(
            jax.ShapeDtypeStruct((NC, C_PAD, F), jnp.float32),
            jax.ShapeDtypeStruct((NC, C_PAD, 16), jnp.float32),
            jax.ShapeDtypeStruct((B,), jnp.int32),
        ),
        scratch_types=[
            pltpu.VMEM((3, CHUNK, F), jnp.float32),
            pltpu.VMEM((MAXQ, CHUNK), jnp.int32),
            pltpu.VMEM((TAIL,), jnp.int32),
            pltpu.VMEM((CHUNK, 16), jnp.float32),
            pltpu.VMEM((B_PER_W,), jnp.int32),
            pltpu.VMEM((B_PER_W,), jnp.int32),
            pltpu.VMEM_SHARED((C_PAD, F), jnp.float32),
            pltpu.VMEM_SHARED((C_PAD, 16), jnp.float32),
            pltpu.SemaphoreType.DMA((3,)),
            pltpu.SemaphoreType.DMA((2,)),
            pltpu.SemaphoreType.DMA((2,)),
        ],
    )
    def sc_segsum(feat_hbm, lblq_hbm, lbl_hbm, idx_hbm, ones_hbm, zg_hbm,
                  zn_hbm, partials_hbm, counts_hbm, targets_hbm,
                  feat_v, lbl_v, ltail_v, ones_v, idx_v, tgt_v,
                  acc_g, acc_n, sem_f, sem_s, sem_o):
        c = lax.axis_index("c")
        s = lax.axis_index("s")
        w = s * NC + c

        zrows = C_PAD // NS
        pltpu.sync_copy(zg_hbm.at[pl.ds(s * zrows, zrows)],
                        acc_g.at[pl.ds(s * zrows, zrows)])
        pltpu.sync_copy(zn_hbm.at[pl.ds(s * zrows, zrows)],
                        acc_n.at[pl.ds(s * zrows, zrows)])

        pltpu.sync_copy(ones_hbm, ones_v)
        pltpu.sync_copy(lblq_hbm.at[w], lbl_v)
        pltpu.sync_copy(idx_hbm.at[pl.ds(w * B_PER_W, B_PER_W)], idx_v)

        def feat_off(i):
            return jnp.minimum(i * NW + w, NFULL - 1) * CHUNK

        def load(i, slot):
            pltpu.async_copy(
                feat_hbm.at[pl.ds(feat_off(i), CHUNK)], feat_v.at[slot],
                sem_f.at[slot])

        def wait_load(i, slot):
            pltpu.make_async_copy(
                feat_hbm.at[pl.ds(feat_off(i), CHUNK)], feat_v.at[slot],
                sem_f.at[slot]).wait()

        def start_scatter(i, slot):
            pltpu.async_copy(
                feat_v.at[slot], acc_g.at[lbl_v.at[i]], sem_s.at[slot],
                add=True)
            pltpu.async_copy(
                ones_v, acc_n.at[lbl_v.at[i]], sem_o.at[slot], add=True)

        def wait_scatter(i, slot):
            pltpu.make_async_copy(
                feat_v.at[slot], acc_g.at[lbl_v.at[i]], sem_s.at[slot]).wait()
            pltpu.make_async_copy(
                ones_v, acc_n.at[lbl_v.at[i]], sem_o.at[slot]).wait()

        load(0, 0)
        load(1, 1)
        load(2, 2)
        plsc.subcore_barrier()
        for i in range(MAXQ):
            slot = i % 3
            wait_load(i, slot)
            start_scatter(i, slot)
            wait_scatter(i, slot)
            if i + 3 < MAXQ:
                load(i + 3, slot)

        @pl.when(w == NW - 1)
        def _():
            off = NFULL * CHUNK
            pltpu.sync_copy(feat_hbm.at[pl.ds(off, TAIL)],
                            feat_v.at[0].at[pl.ds(0, TAIL)])
            pltpu.sync_copy(lbl_hbm.at[pl.ds(off, TAIL)], ltail_v)
            pltpu.sync_copy(feat_v.at[0].at[pl.ds(0, TAIL)],
                            acc_g.at[ltail_v], add=True)
            pltpu.sync_copy(ones_v.at[pl.ds(0, TAIL)],
                            acc_n.at[ltail_v], add=True)

        pltpu.sync_copy(lbl_hbm.at[idx_v], tgt_v)
        pltpu.sync_copy(tgt_v, targets_hbm.at[pl.ds(w * B_PER_W, B_PER_W)])

        plsc.subcore_barrier()

        @pl.when(s == 0)
        def _():
            pltpu.sync_copy(acc_g, partials_hbm.at[c])
            pltpu.sync_copy(acc_n, counts_hbm.at[c])

    return sc_segsum


_BLK = 1024
_GRID = B // _BLK


def _tc_body(res_ref, part_ref, cnt_ref, tgt_ref, out_ref, g_ref, nums_ref,
             acc):
    i = pl.program_id(0)

    @pl.when(i == 0)
    def _():
        g_ref[...] = part_ref[0] + part_ref[1]
        cnt = cnt_ref[0] + cnt_ref[1]
        sel = jnp.full((1, 16), 1.0 / 16.0, jnp.float32)
        nums_ref[...] = lax.dot_general(
            sel, cnt, (((1,), (1,)), ((), ())),
            preferred_element_type=jnp.float32,
            precision=lax.Precision.HIGHEST)
        acc[0] = 0.0

    x = res_ref[...]
    norm = jnp.sqrt(jnp.sum(x * x, axis=1, keepdims=True))
    x = x / jnp.maximum(norm, 1e-12)

    vec = lax.dot_general(x, g_ref[...], (((1,), (1,)), ((), ())),
                          preferred_element_type=jnp.float32)

    nums = nums_ref[...]
    col = lax.broadcasted_iota(jnp.int32, (1, C_PAD), 1)
    mask = jnp.where((nums > 0.0) & (col < C), 1.0, 0.0)
    scale = 1.0 / (TEMP * jnp.maximum(nums, 1.0))

    mexp = jnp.exp(vec * scale) * mask
    sums = jnp.sum(mexp, axis=1, keepdims=True) + 1e-6

    t = tgt_ref[...]
    hit = lax.broadcasted_iota(jnp.int32, (_BLK, C_PAD), 1) == t
    ex_t = jnp.sum(jnp.where(hit, mexp, 0.0), axis=1, keepdims=True)
    picked = jnp.log(ex_t + 1e-6 * sums) - jnp.log(sums)

    acc[0] += jnp.sum(picked)

    @pl.when(i == _GRID - 1)
    def _():
        out_ref[...] = jnp.full((1, 1), -acc[0] / float(B), jnp.float32)


def _tc_loss(results, partials, counts, targets_col):
    return pl.pallas_call(
        _tc_body,
        grid=(_GRID,),
        in_specs=[
            pl.BlockSpec((_BLK, F), lambda i: (i, 0)),
            pl.BlockSpec((NC, C_PAD, F), lambda i: (0, 0, 0)),
            pl.BlockSpec((NC, C_PAD, 16), lambda i: (0, 0, 0)),
            pl.BlockSpec((_BLK, 1), lambda i: (i, 0)),
        ],
        out_specs=pl.BlockSpec((1, 1), lambda i: (0, 0)),
        out_shape=jax.ShapeDtypeStruct((1, 1), jnp.float32),
        scratch_shapes=[
            pltpu.VMEM((C_PAD, F), jnp.float32),
            pltpu.VMEM((1, C_PAD), jnp.float32),
            pltpu.SMEM((1,), jnp.float32),
        ],
    )(results, partials, counts, targets_col)


def kernel(results, indexes, features, labels):
    ones_rows = jnp.ones((CHUNK, 16), jnp.float32)
    zg = jnp.zeros((C_PAD, F), jnp.float32)
    zn = jnp.zeros((C_PAD, 16), jnp.float32)
    npad = MAXQ * NW * CHUNK - NFULL * CHUNK
    dummy = C + (jnp.arange(npad, dtype=jnp.int32) % (C_PAD - C))
    lbl_pad = jnp.concatenate([labels[:NFULL * CHUNK], dummy])
    lblq = lbl_pad.reshape(MAXQ, NW, CHUNK).transpose(1, 0, 2)

    partials, counts, targets = _build_sc_segsum()(
        features, lblq, labels, indexes, ones_rows, zg, zn)

    targets_col = targets.reshape(B, 1)

    loss = _tc_loss(results, partials, counts, targets_col)
    return loss.reshape(())
